# Initial kernel scaffold; baseline (speedup 1.0000x reference)
import functools

import jax
import jax.numpy as jnp
from jax import lax
from jax.experimental import pallas as pl
from jax.experimental.pallas import tpu as pltpu
from jax.experimental.pallas import tpu_sc as plsc

N = 10000
E = 160000
R = 17
F_IN = 128
HID = 64
NCLS = 16

NC, NS, L = 2, 16, 16
NW = NC * NS
E2 = 2 * E
EPW = E2 // NW
B = 400
NCHUNK = EPW // B

DEG_SLICE = 10640
DEG_PAD = NS * DEG_SLICE
ROWS_PT = N // NS

_MESH = plsc.VectorSubcoreMesh(core_axis_name="c", subcore_axis_name="s")
_MB = 400


@functools.partial(
    pl.kernel,
    out_type=jax.ShapeDtypeStruct((NC, DEG_PAD), jnp.float32),
    mesh=_MESH,
    scratch_types=[
        pltpu.VMEM((B,), jnp.int32),
        pltpu.VMEM((B,), jnp.float32),
        pltpu.VMEM_SHARED((DEG_PAD,), jnp.float32),
    ],
)
def _deg_kernel(didx, zeros, deg_out, idx_v, ones_v, deg_sh):
    cid = lax.axis_index("c")
    sid = lax.axis_index("s")
    wid = sid * NC + cid
    pltpu.sync_copy(zeros.at[pl.ds(sid * DEG_SLICE, DEG_SLICE)],
                    deg_sh.at[pl.ds(sid * DEG_SLICE, DEG_SLICE)])
    one = jnp.full((L,), 1.0, jnp.float32)
    for i in range(B // L):
        ones_v[pl.ds(i * L, L)] = one
    plsc.subcore_barrier()
    base = wid * EPW

    def chunk(i, carry):
        pltpu.sync_copy(didx.at[pl.ds(base + i * B, B)], idx_v)
        pltpu.sync_copy(ones_v, deg_sh.at[idx_v], add=True)
        return carry

    lax.fori_loop(0, NCHUNK, chunk, 0)
    plsc.subcore_barrier()
    pltpu.sync_copy(deg_sh.at[pl.ds(sid * DEG_SLICE, DEG_SLICE)],
                    deg_out.at[cid, pl.ds(sid * DEG_SLICE, DEG_SLICE)])


def _make_agg(width):
    @functools.partial(
        pl.kernel,
        out_type=jax.ShapeDtypeStruct((NC, N, width), jnp.float32),
        mesh=_MESH,
        scratch_types=[
            pltpu.VMEM((B,), jnp.int32),
            pltpu.VMEM((B,), jnp.int32),
            pltpu.VMEM((B,), jnp.int32),
            pltpu.VMEM((B,), jnp.float32),
            pltpu.VMEM((B, width), jnp.float32),
            pltpu.VMEM_SHARED((N, width), jnp.float32),
            pltpu.SemaphoreType.DMA,
            pltpu.SemaphoreType.DMA,
        ],
    )
    def agg(table, gidx, sidx, didx, deg, zeros, out,
            gidx_v, sidx_v, didx_v, deg_v, rows_v, acc_sh, sem_r, sem_d):
        cid = lax.axis_index("c")
        sid = lax.axis_index("s")
        wid = sid * NC + cid
        pltpu.sync_copy(zeros.at[pl.ds(sid * ROWS_PT, ROWS_PT)],
                        acc_sh.at[pl.ds(sid * ROWS_PT, ROWS_PT)])
        plsc.subcore_barrier()
        base = wid * EPW

        def chunk(i, carry):
            off = base + i * B
            pltpu.sync_copy(gidx.at[pl.ds(off, B)], gidx_v)
            pltpu.sync_copy(didx.at[pl.ds(off, B)], didx_v)
            pltpu.sync_copy(sidx.at[pl.ds(off, B)], sidx_v)
            rcp = pltpu.async_copy(table.at[gidx_v], rows_v, sem_r)
            dcp = pltpu.async_copy(deg.at[didx_v], deg_v, sem_d)
            dcp.wait()
            rcp.wait()

            def group(g, c2):
                d16 = deg_v[pl.ds(g * L, L)]
                v16 = 1.0 / d16
                eids = lax.iota(jnp.int32, L) + lax.broadcast(g * L, (L,))
                for j in range(width):
                    jj = jnp.full((L,), j, jnp.int32)
                    col = plsc.load_gather(rows_v, [eids, jj])
                    plsc.store_scatter(rows_v, [eids, jj], col * v16)
                return c2

            lax.fori_loop(0, B // L, group, 0)
            pltpu.sync_copy(rows_v, acc_sh.at[sidx_v], add=True)
            return carry

        lax.fori_loop(0, NCHUNK, chunk, 0)
        plsc.subcore_barrier()
        pltpu.sync_copy(acc_sh.at[pl.ds(sid * ROWS_PT, ROWS_PT)],
                        out.at[cid, pl.ds(sid * ROWS_PT, ROWS_PT)])

    return agg


_agg64 = _make_agg(HID)
_agg16 = _make_agg(NCLS)


def _mm1_body(x_ref, w_ref, o_ref):
    o_ref[...] = jnp.dot(x_ref[...], w_ref[...],
                         preferred_element_type=jnp.float32)


def _l2_body(a0_ref, a1_ref, s1_ref, b1_ref, w_ref, o_ref):
    h = jnp.maximum(a0_ref[0] + a1_ref[0] + s1_ref[...] + b1_ref[...], 0.0)
    o_ref[...] = jnp.dot(h, w_ref[...], preferred_element_type=jnp.float32)


def _comb_body(a0_ref, a1_ref, s2_ref, b2_ref, o_ref):
    o_ref[...] = a0_ref[0] + a1_ref[0] + s2_ref[...] + b2_ref[...]


def kernel(features, triples, weights1, bias1, weights2, bias2):
    fr = triples[:, 0]
    rel = triples[:, 1]
    to = triples[:, 2]
    didx = jnp.concatenate([fr * R + rel, to * R + rel + 8])
    gidx = jnp.concatenate([to * R + rel, fr * R + rel + 8])
    sidx = jnp.concatenate([fr, to])
    w1cat = jnp.transpose(weights1, (1, 0, 2)).reshape(F_IN, R * HID)
    w2cat = jnp.transpose(weights2, (1, 0, 2)).reshape(HID, R * NCLS)
    zeros_deg = jnp.zeros((DEG_PAD,), jnp.float32)
    zeros1 = jnp.zeros((N, HID), jnp.float32)
    zeros2 = jnp.zeros((N, NCLS), jnp.float32)

    degp = _deg_kernel(didx, zeros_deg)
    deg = degp[0] + degp[1]

    nb = N // _MB
    table1 = pl.pallas_call(
        _mm1_body,
        grid=(nb,),
        in_specs=[pl.BlockSpec((_MB, F_IN), lambda i: (i, 0)),
                  pl.BlockSpec((F_IN, R * HID), lambda i: (0, 0))],
        out_specs=pl.BlockSpec((_MB, R * HID), lambda i: (i, 0)),
        out_shape=jax.ShapeDtypeStruct((N, R * HID), jnp.float32),
    )(features, w1cat)

    acc1p = _agg64(table1.reshape(N * R, HID), gidx, sidx, didx, deg, zeros1)

    self1 = table1[:, 16 * HID:]
    table2 = pl.pallas_call(
        _l2_body,
        grid=(nb,),
        in_specs=[pl.BlockSpec((1, _MB, HID), lambda i: (0, i, 0)),
                  pl.BlockSpec((1, _MB, HID), lambda i: (1, i, 0)),
                  pl.BlockSpec((_MB, HID), lambda i: (i, 0)),
                  pl.BlockSpec((1, HID), lambda i: (0, 0)),
                  pl.BlockSpec((HID, R * NCLS), lambda i: (0, 0))],
        out_specs=pl.BlockSpec((_MB, R * NCLS), lambda i: (i, 0)),
        out_shape=jax.ShapeDtypeStruct((N, R * NCLS), jnp.float32),
    )(acc1p, acc1p, self1, bias1.reshape(1, HID), w2cat)

    acc2p = _agg16(table2.reshape(N * R, NCLS), gidx, sidx, didx, deg, zeros2)

    self2 = table2[:, 16 * NCLS:]
    out = pl.pallas_call(
        _comb_body,
        grid=(nb,),
        in_specs=[pl.BlockSpec((1, _MB, NCLS), lambda i: (0, i, 0)),
                  pl.BlockSpec((1, _MB, NCLS), lambda i: (1, i, 0)),
                  pl.BlockSpec((_MB, NCLS), lambda i: (i, 0)),
                  pl.BlockSpec((1, NCLS), lambda i: (0, 0))],
        out_specs=pl.BlockSpec((_MB, NCLS), lambda i: (i, 0)),
        out_shape=jax.ShapeDtypeStruct((N, NCLS), jnp.float32),
    )(acc2p, acc2p, self2, bias2.reshape(1, NCLS))
    return out


# speedup vs baseline: 4.9316x; 4.9316x over previous
import functools

import jax
import jax.numpy as jnp
from jax import lax
from jax.experimental import pallas as pl
from jax.experimental.pallas import tpu as pltpu
from jax.experimental.pallas import tpu_sc as plsc

N = 10000
E = 160000
R = 17
F_IN = 128
HID = 64
NCLS = 16

NC, NS, L = 2, 16, 16
NW = NC * NS
E2 = 2 * E
EPW = E2 // NW
B = 400
NCHUNK = EPW // B

DEG_SLICE = 10752
DEG_PAD = NS * DEG_SLICE
N_PAD = 10240
ROWS_PT = N_PAD // NS

_MESH = plsc.VectorSubcoreMesh(core_axis_name="c", subcore_axis_name="s")
_MB = 400


@functools.partial(
    pl.kernel,
    out_type=jax.ShapeDtypeStruct((NC * DEG_PAD,), jnp.float32),
    mesh=_MESH,
    compiler_params=pltpu.CompilerParams(needs_layout_passes=False, use_tc_tiling_on_sc=False),
    scratch_types=[
        pltpu.VMEM((B,), jnp.int32),
        pltpu.VMEM((B,), jnp.float32),
        pltpu.VMEM_SHARED((DEG_PAD,), jnp.float32),
    ],
)
def _deg_kernel(didx, zeros, deg_out, idx_v, ones_v, deg_sh):
    cid = lax.axis_index("c")
    sid = lax.axis_index("s")
    wid = sid * NC + cid
    pltpu.sync_copy(zeros.at[pl.ds(sid * DEG_SLICE, DEG_SLICE)],
                    deg_sh.at[pl.ds(sid * DEG_SLICE, DEG_SLICE)])
    one = jnp.full((L,), 1.0, jnp.float32)
    for i in range(B // L):
        ones_v[pl.ds(i * L, L)] = one
    plsc.subcore_barrier()
    base = wid * EPW

    def chunk(i, carry):
        pltpu.sync_copy(didx.at[pl.ds(base + i * B, B)], idx_v)
        pltpu.sync_copy(ones_v, deg_sh.at[idx_v], add=True)
        return carry

    lax.fori_loop(0, NCHUNK, chunk, 0)
    plsc.subcore_barrier()
    pltpu.sync_copy(deg_sh.at[pl.ds(sid * DEG_SLICE, DEG_SLICE)],
                    deg_out.at[pl.ds(cid * DEG_PAD + sid * DEG_SLICE, DEG_SLICE)])


def _make_agg(width):
    @functools.partial(
        pl.kernel,
        out_type=jax.ShapeDtypeStruct((NC, N_PAD, width), jnp.float32),
        mesh=_MESH,
        compiler_params=pltpu.CompilerParams(needs_layout_passes=False, use_tc_tiling_on_sc=False),
        scratch_types=[
            pltpu.VMEM((B,), jnp.int32),
            pltpu.VMEM((B,), jnp.int32),
            pltpu.VMEM((B,), jnp.int32),
            pltpu.VMEM((B,), jnp.float32),
            pltpu.VMEM((B, width), jnp.float32),
            pltpu.VMEM_SHARED((N_PAD, width), jnp.float32),
            pltpu.SemaphoreType.DMA,
            pltpu.SemaphoreType.DMA,
        ],
    )
    def agg(table, gidx, sidx, didx, deg, zeros, out,
            gidx_v, sidx_v, didx_v, deg_v, rows_v, acc_sh, sem_r, sem_d):
        cid = lax.axis_index("c")
        sid = lax.axis_index("s")
        wid = sid * NC + cid
        pltpu.sync_copy(zeros.at[pl.ds(sid * ROWS_PT, ROWS_PT)],
                        acc_sh.at[pl.ds(sid * ROWS_PT, ROWS_PT)])
        plsc.subcore_barrier()
        base = wid * EPW

        def chunk(i, carry):
            off = base + i * B
            pltpu.sync_copy(gidx.at[pl.ds(off, B)], gidx_v)
            pltpu.sync_copy(didx.at[pl.ds(off, B)], didx_v)
            pltpu.sync_copy(sidx.at[pl.ds(off, B)], sidx_v)
            rcp = pltpu.async_copy(table.at[gidx_v], rows_v, sem_r)
            dcp = pltpu.async_copy(deg.at[didx_v], deg_v, sem_d)
            dcp.wait()
            rcp.wait()

            def group(g, c2):
                d16 = deg_v[pl.ds(g * L, L)]
                v16 = 1.0 / d16
                eids = lax.iota(jnp.int32, L) + lax.broadcast(g * L, (L,))
                for j in range(width):
                    jj = jnp.full((L,), j, jnp.int32)
                    col = plsc.load_gather(rows_v, [eids, jj])
                    plsc.store_scatter(rows_v, [eids, jj], col * v16)
                return c2

            lax.fori_loop(0, B // L, group, 0)
            pltpu.sync_copy(rows_v, acc_sh.at[sidx_v], add=True)
            return carry

        lax.fori_loop(0, NCHUNK, chunk, 0)
        plsc.subcore_barrier()
        pltpu.sync_copy(acc_sh.at[pl.ds(sid * ROWS_PT, ROWS_PT)],
                        out.at[cid, pl.ds(sid * ROWS_PT, ROWS_PT)])

    return agg


_agg64 = _make_agg(HID)
_agg16 = _make_agg(NCLS)


def _mm1_body(x_ref, w_ref, o_ref):
    o_ref[...] = jnp.dot(x_ref[...], w_ref[...],
                         preferred_element_type=jnp.float32)


def _l2_body(a0_ref, a1_ref, s1_ref, b1_ref, w_ref, o_ref):
    h = jnp.maximum(a0_ref[0] + a1_ref[0] + s1_ref[...] + b1_ref[...], 0.0)
    o_ref[...] = jnp.dot(h, w_ref[...], preferred_element_type=jnp.float32)


def _comb_body(a0_ref, a1_ref, s2_ref, b2_ref, o_ref):
    o_ref[...] = a0_ref[0] + a1_ref[0] + s2_ref[...] + b2_ref[...]


def kernel(features, triples, weights1, bias1, weights2, bias2):
    fr = triples[:, 0]
    rel = triples[:, 1]
    to = triples[:, 2]
    didx = jnp.concatenate([fr * R + rel, to * R + rel + 8])
    gidx = jnp.concatenate([to * R + rel, fr * R + rel + 8])
    sidx = jnp.concatenate([fr, to])
    w1cat = jnp.transpose(weights1, (1, 0, 2)).reshape(F_IN, R * HID)
    w2cat = jnp.transpose(weights2, (1, 0, 2)).reshape(HID, R * NCLS)
    zeros_deg = jnp.zeros((DEG_PAD,), jnp.float32)
    zeros1 = jnp.zeros((N_PAD, HID), jnp.float32)
    zeros2 = jnp.zeros((N_PAD, NCLS), jnp.float32)

    degp = _deg_kernel(didx, zeros_deg).reshape(NC, DEG_PAD)
    deg = degp[0] + degp[1]

    nb = N // _MB
    table1 = pl.pallas_call(
        _mm1_body,
        grid=(nb,),
        in_specs=[pl.BlockSpec((_MB, F_IN), lambda i: (i, 0)),
                  pl.BlockSpec((F_IN, R * HID), lambda i: (0, 0))],
        out_specs=pl.BlockSpec((_MB, R * HID), lambda i: (i, 0)),
        out_shape=jax.ShapeDtypeStruct((N, R * HID), jnp.float32),
    )(features, w1cat)

    acc1p = _agg64(table1.reshape(N * R, HID), gidx, sidx, didx, deg, zeros1)

    self1 = table1[:, 16 * HID:]
    table2 = pl.pallas_call(
        _l2_body,
        grid=(nb,),
        in_specs=[pl.BlockSpec((1, _MB, HID), lambda i: (0, i, 0)),
                  pl.BlockSpec((1, _MB, HID), lambda i: (1, i, 0)),
                  pl.BlockSpec((_MB, HID), lambda i: (i, 0)),
                  pl.BlockSpec((1, HID), lambda i: (0, 0)),
                  pl.BlockSpec((HID, R * NCLS), lambda i: (0, 0))],
        out_specs=pl.BlockSpec((_MB, R * NCLS), lambda i: (i, 0)),
        out_shape=jax.ShapeDtypeStruct((N, R * NCLS), jnp.float32),
    )(acc1p, acc1p, self1, bias1.reshape(1, HID), w2cat)

    acc2p = _agg16(table2.reshape(N * R, NCLS), gidx, sidx, didx, deg, zeros2)

    self2 = table2[:, 16 * NCLS:]
    out = pl.pallas_call(
        _comb_body,
        grid=(nb,),
        in_specs=[pl.BlockSpec((1, _MB, NCLS), lambda i: (0, i, 0)),
                  pl.BlockSpec((1, _MB, NCLS), lambda i: (1, i, 0)),
                  pl.BlockSpec((_MB, NCLS), lambda i: (i, 0)),
                  pl.BlockSpec((1, NCLS), lambda i: (0, 0))],
        out_specs=pl.BlockSpec((_MB, NCLS), lambda i: (i, 0)),
        out_shape=jax.ShapeDtypeStruct((N, NCLS), jnp.float32),
    )(acc2p, acc2p, self2, bias2.reshape(1, NCLS))
    return out


# speedup vs baseline: 5.4639x; 1.1079x over previous
import functools

import jax
import jax.numpy as jnp
from jax import lax
from jax.experimental import pallas as pl
from jax.experimental.pallas import tpu as pltpu
from jax.experimental.pallas import tpu_sc as plsc

N = 10000
E = 160000
R = 17
F_IN = 128
HID = 64
NCLS = 16

NC, NS, L = 2, 16, 16
NW = NC * NS
E2 = 2 * E
EPW = E2 // NW
B = 400
NCHUNK = EPW // B

DEG_SLICE = 10752
DEG_PAD = NS * DEG_SLICE
N_PAD = 10240
ROWS_PT = N_PAD // NS

_MESH = plsc.VectorSubcoreMesh(core_axis_name="c", subcore_axis_name="s")
_MB = 400


@functools.partial(
    pl.kernel,
    out_type=jax.ShapeDtypeStruct((NC * DEG_PAD,), jnp.float32),
    mesh=_MESH,
    compiler_params=pltpu.CompilerParams(needs_layout_passes=False, use_tc_tiling_on_sc=False),
    scratch_types=[
        pltpu.VMEM((B,), jnp.int32),
        pltpu.VMEM((B,), jnp.float32),
        pltpu.VMEM_SHARED((DEG_PAD,), jnp.float32),
    ],
)
def _deg_kernel(didx, zeros, deg_out, idx_v, ones_v, deg_sh):
    cid = lax.axis_index("c")
    sid = lax.axis_index("s")
    wid = sid * NC + cid
    pltpu.sync_copy(zeros.at[pl.ds(sid * DEG_SLICE, DEG_SLICE)],
                    deg_sh.at[pl.ds(sid * DEG_SLICE, DEG_SLICE)])
    one = jnp.full((L,), 1.0, jnp.float32)
    for i in range(B // L):
        ones_v[pl.ds(i * L, L)] = one
    plsc.subcore_barrier()
    base = wid * EPW

    def chunk(i, carry):
        pltpu.sync_copy(didx.at[pl.ds(base + i * B, B)], idx_v)
        pltpu.sync_copy(ones_v, deg_sh.at[idx_v], add=True)
        return carry

    lax.fori_loop(0, NCHUNK, chunk, 0)
    plsc.subcore_barrier()
    pltpu.sync_copy(deg_sh.at[pl.ds(sid * DEG_SLICE, DEG_SLICE)],
                    deg_out.at[pl.ds(cid * DEG_PAD + sid * DEG_SLICE, DEG_SLICE)])


def _make_agg(width):
    @functools.partial(
        pl.kernel,
        out_type=jax.ShapeDtypeStruct((NC, N_PAD, width), jnp.float32),
        mesh=_MESH,
        compiler_params=pltpu.CompilerParams(needs_layout_passes=False, use_tc_tiling_on_sc=False),
        scratch_types=[
            pltpu.VMEM((NCHUNK, B), jnp.int32),
            pltpu.VMEM((NCHUNK, B), jnp.int32),
            pltpu.VMEM((NCHUNK, B), jnp.int32),
            pltpu.VMEM((2, B), jnp.float32),
            pltpu.VMEM((2, B, width), jnp.float32),
            pltpu.VMEM_SHARED((N_PAD, width), jnp.float32),
            pltpu.SemaphoreType.DMA,
            pltpu.SemaphoreType.DMA,
            pltpu.SemaphoreType.DMA,
            pltpu.SemaphoreType.DMA,
        ],
    )
    def agg(table, gidx, sidx, didx, deg, zeros, out,
            gidx_v, sidx_v, didx_v, deg_v, rows_v, acc_sh,
            sem_r0, sem_r1, sem_d0, sem_d1):
        cid = lax.axis_index("c")
        sid = lax.axis_index("s")
        wid = sid * NC + cid
        pltpu.sync_copy(zeros.at[pl.ds(sid * ROWS_PT, ROWS_PT)],
                        acc_sh.at[pl.ds(sid * ROWS_PT, ROWS_PT)])
        pltpu.sync_copy(gidx.at[wid], gidx_v)
        pltpu.sync_copy(sidx.at[wid], sidx_v)
        pltpu.sync_copy(didx.at[wid], didx_v)
        plsc.subcore_barrier()
        sem_r = (sem_r0, sem_r1)
        sem_d = (sem_d0, sem_d1)

        def fire(i, b):
            pltpu.async_copy(table.at[gidx_v.at[i]], rows_v.at[b], sem_r[b])
            pltpu.async_copy(deg.at[didx_v.at[i]], deg_v.at[b], sem_d[b])

        def wait(b):
            pltpu.make_async_copy(table.at[gidx_v.at[0]], rows_v.at[b],
                                  sem_r[b]).wait()
            pltpu.make_async_copy(deg.at[didx_v.at[0]], deg_v.at[b],
                                  sem_d[b]).wait()

        def scale_and_scatter(i, b):
            rows_b = rows_v.at[b]
            deg_b = deg_v.at[b]

            def group(g, c2):
                d16 = deg_b[pl.ds(g * L, L)]
                v16 = 1.0 / d16
                eids = lax.iota(jnp.int32, L) + lax.broadcast(g * L, (L,))
                for j in range(width):
                    jj = jnp.full((L,), j, jnp.int32)
                    col = plsc.load_gather(rows_b, [eids, jj])
                    plsc.store_scatter(rows_b, [eids, jj], col * v16)
                return c2

            lax.fori_loop(0, B // L, group, 0)
            pltpu.sync_copy(rows_b, acc_sh.at[sidx_v.at[i]], add=True)

        fire(0, 0)

        def pair(p, carry):
            i0 = 2 * p
            fire(i0 + 1, 1)
            wait(0)
            scale_and_scatter(i0, 0)
            fire(i0 + 2, 0)
            wait(1)
            scale_and_scatter(i0 + 1, 1)
            return carry

        lax.fori_loop(0, (NCHUNK - 1) // 2, pair, 0)
        wait(0)
        scale_and_scatter(NCHUNK - 1, 0)
        plsc.subcore_barrier()
        pltpu.sync_copy(acc_sh.at[pl.ds(sid * ROWS_PT, ROWS_PT)],
                        out.at[cid, pl.ds(sid * ROWS_PT, ROWS_PT)])

    return agg


_agg64 = _make_agg(HID)
_agg16 = _make_agg(NCLS)


def _mm1_body(x_ref, w_ref, o_ref):
    o_ref[...] = jnp.dot(x_ref[...], w_ref[...],
                         preferred_element_type=jnp.float32)


def _l2_body(a0_ref, a1_ref, s1_ref, b1_ref, w_ref, o_ref):
    h = jnp.maximum(a0_ref[0] + a1_ref[0] + s1_ref[...] + b1_ref[...], 0.0)
    o_ref[...] = jnp.dot(h, w_ref[...], preferred_element_type=jnp.float32)


def _comb_body(a0_ref, a1_ref, s2_ref, b2_ref, o_ref):
    o_ref[...] = a0_ref[0] + a1_ref[0] + s2_ref[...] + b2_ref[...]


def kernel(features, triples, weights1, bias1, weights2, bias2):
    fr = triples[:, 0]
    rel = triples[:, 1]
    to = triples[:, 2]
    didx = jnp.concatenate([fr * R + rel, to * R + rel + 8])
    gidx = jnp.concatenate([to * R + rel, fr * R + rel + 8])
    sidx = jnp.concatenate([fr, to])
    gidx3 = gidx.reshape(NW, NCHUNK, B)
    sidx3 = sidx.reshape(NW, NCHUNK, B)
    didx3 = didx.reshape(NW, NCHUNK, B)
    w1cat = jnp.transpose(weights1, (1, 0, 2)).reshape(F_IN, R * HID)
    w2cat = jnp.transpose(weights2, (1, 0, 2)).reshape(HID, R * NCLS)
    zeros_deg = jnp.zeros((DEG_PAD,), jnp.float32)
    zeros1 = jnp.zeros((N_PAD, HID), jnp.float32)
    zeros2 = jnp.zeros((N_PAD, NCLS), jnp.float32)

    degp = _deg_kernel(didx, zeros_deg).reshape(NC, DEG_PAD)
    deg = degp[0] + degp[1]

    nb = N // _MB
    table1 = pl.pallas_call(
        _mm1_body,
        grid=(nb,),
        in_specs=[pl.BlockSpec((_MB, F_IN), lambda i: (i, 0)),
                  pl.BlockSpec((F_IN, R * HID), lambda i: (0, 0))],
        out_specs=pl.BlockSpec((_MB, R * HID), lambda i: (i, 0)),
        out_shape=jax.ShapeDtypeStruct((N, R * HID), jnp.float32),
    )(features, w1cat)

    acc1p = _agg64(table1.reshape(N * R, HID), gidx3, sidx3, didx3, deg, zeros1)

    self1 = table1[:, 16 * HID:]
    table2 = pl.pallas_call(
        _l2_body,
        grid=(nb,),
        in_specs=[pl.BlockSpec((1, _MB, HID), lambda i: (0, i, 0)),
                  pl.BlockSpec((1, _MB, HID), lambda i: (1, i, 0)),
                  pl.BlockSpec((_MB, HID), lambda i: (i, 0)),
                  pl.BlockSpec((1, HID), lambda i: (0, 0)),
                  pl.BlockSpec((HID, R * NCLS), lambda i: (0, 0))],
        out_specs=pl.BlockSpec((_MB, R * NCLS), lambda i: (i, 0)),
        out_shape=jax.ShapeDtypeStruct((N, R * NCLS), jnp.float32),
    )(acc1p, acc1p, self1, bias1.reshape(1, HID), w2cat)

    acc2p = _agg16(table2.reshape(N * R, NCLS), gidx3, sidx3, didx3, deg, zeros2)

    self2 = table2[:, 16 * NCLS:]
    out = pl.pallas_call(
        _comb_body,
        grid=(nb,),
        in_specs=[pl.BlockSpec((1, _MB, NCLS), lambda i: (0, i, 0)),
                  pl.BlockSpec((1, _MB, NCLS), lambda i: (1, i, 0)),
                  pl.BlockSpec((_MB, NCLS), lambda i: (i, 0)),
                  pl.BlockSpec((1, NCLS), lambda i: (0, 0))],
        out_specs=pl.BlockSpec((_MB, NCLS), lambda i: (i, 0)),
        out_shape=jax.ShapeDtypeStruct((N, NCLS), jnp.float32),
    )(acc2p, acc2p, self2, bias2.reshape(1, NCLS))
    return out


# speedup vs baseline: 19.0074x; 3.4787x over previous
import functools

import jax
import jax.numpy as jnp
from jax import lax
from jax.experimental import pallas as pl
from jax.experimental.pallas import tpu as pltpu
from jax.experimental.pallas import tpu_sc as plsc

N = 10000
E = 160000
R = 17
F_IN = 128
HID = 64
NCLS = 16

NC, NS, L = 2, 16, 16
NW = NC * NS
E2 = 2 * E
EPW = E2 // NW
B = 400
NCHUNK = EPW // B

DEG_SLICE = 10752
DEG_PAD = NS * DEG_SLICE
N_PAD = 10240
ROWS_PT = N_PAD // NS

_MESH = plsc.VectorSubcoreMesh(core_axis_name="c", subcore_axis_name="s")
_MB = 400


@functools.partial(
    pl.kernel,
    out_type=jax.ShapeDtypeStruct((NC * DEG_PAD,), jnp.float32),
    mesh=_MESH,
    compiler_params=pltpu.CompilerParams(needs_layout_passes=False, use_tc_tiling_on_sc=False),
    scratch_types=[
        pltpu.VMEM((B,), jnp.int32),
        pltpu.VMEM((B,), jnp.float32),
        pltpu.VMEM_SHARED((DEG_PAD,), jnp.float32),
    ],
)
def _deg_kernel(didx, zeros, deg_out, idx_v, ones_v, deg_sh):
    cid = lax.axis_index("c")
    sid = lax.axis_index("s")
    wid = sid * NC + cid
    pltpu.sync_copy(zeros.at[pl.ds(sid * DEG_SLICE, DEG_SLICE)],
                    deg_sh.at[pl.ds(sid * DEG_SLICE, DEG_SLICE)])
    one = jnp.full((L,), 1.0, jnp.float32)
    for i in range(B // L):
        ones_v[pl.ds(i * L, L)] = one
    plsc.subcore_barrier()
    base = wid * EPW

    def chunk(i, carry):
        pltpu.sync_copy(didx.at[pl.ds(base + i * B, B)], idx_v)
        pltpu.sync_copy(ones_v, deg_sh.at[idx_v], add=True)
        return carry

    lax.fori_loop(0, NCHUNK, chunk, 0)
    plsc.subcore_barrier()
    pltpu.sync_copy(deg_sh.at[pl.ds(sid * DEG_SLICE, DEG_SLICE)],
                    deg_out.at[pl.ds(cid * DEG_PAD + sid * DEG_SLICE, DEG_SLICE)])


def _make_agg(width):
    @functools.partial(
        pl.kernel,
        out_type=jax.ShapeDtypeStruct((NC, N_PAD, width), jnp.float32),
        mesh=_MESH,
        compiler_params=pltpu.CompilerParams(needs_layout_passes=False, use_tc_tiling_on_sc=False),
        scratch_types=[
            pltpu.VMEM((NCHUNK, B), jnp.int32),
            pltpu.VMEM((NCHUNK, B), jnp.int32),
            pltpu.VMEM((NCHUNK, B), jnp.int32),
            pltpu.VMEM((2, B), jnp.float32),
            pltpu.VMEM((2, B, width), jnp.float32),
            pltpu.VMEM_SHARED((N_PAD, width), jnp.float32),
            pltpu.SemaphoreType.DMA,
            pltpu.SemaphoreType.DMA,
            pltpu.SemaphoreType.DMA,
            pltpu.SemaphoreType.DMA,
        ],
    )
    def agg(table, gidx, sidx, didx, deg, zeros, out,
            gidx_v, sidx_v, didx_v, deg_v, rows_v, acc_sh,
            sem_r0, sem_r1, sem_d0, sem_d1):
        cid = lax.axis_index("c")
        sid = lax.axis_index("s")
        wid = sid * NC + cid
        pltpu.sync_copy(zeros.at[pl.ds(sid * ROWS_PT, ROWS_PT)],
                        acc_sh.at[pl.ds(sid * ROWS_PT, ROWS_PT)])
        pltpu.sync_copy(gidx.at[wid], gidx_v)
        pltpu.sync_copy(sidx.at[wid], sidx_v)
        pltpu.sync_copy(didx.at[wid], didx_v)
        plsc.subcore_barrier()
        sem_r = (sem_r0, sem_r1)
        sem_d = (sem_d0, sem_d1)

        def fire(i, b):
            pltpu.async_copy(table.at[gidx_v.at[i]], rows_v.at[b], sem_r[b])
            pltpu.async_copy(deg.at[didx_v.at[i]], deg_v.at[b], sem_d[b])

        def wait(b):
            pltpu.make_async_copy(table.at[gidx_v.at[0]], rows_v.at[b],
                                  sem_r[b]).wait()
            pltpu.make_async_copy(deg.at[didx_v.at[0]], deg_v.at[b],
                                  sem_d[b]).wait()

        def scale_and_scatter(i, b):
            rows_b = rows_v.at[b]
            deg_b = deg_v.at[b]
            @plsc.parallel_loop(0, B // L, unroll=5)
            def scale(g):
                v16 = 1.0 / deg_b[pl.ds(g * L, L)]
                for l in range(L):
                    vb = lax.broadcast(v16[l], (L,))
                    e = g * L + l
                    for q in range(width // L):
                        rows_b[e, pl.ds(q * L, L)] = (
                            rows_b[e, pl.ds(q * L, L)] * vb)

            pltpu.sync_copy(rows_b, acc_sh.at[sidx_v.at[i]], add=True)

        fire(0, 0)

        def pair(p, carry):
            i0 = 2 * p
            fire(i0 + 1, 1)
            wait(0)
            scale_and_scatter(i0, 0)
            fire(i0 + 2, 0)
            wait(1)
            scale_and_scatter(i0 + 1, 1)
            return carry

        lax.fori_loop(0, (NCHUNK - 1) // 2, pair, 0)
        wait(0)
        scale_and_scatter(NCHUNK - 1, 0)
        plsc.subcore_barrier()
        pltpu.sync_copy(acc_sh.at[pl.ds(sid * ROWS_PT, ROWS_PT)],
                        out.at[cid, pl.ds(sid * ROWS_PT, ROWS_PT)])

    return agg


_agg64 = _make_agg(HID)
_agg16 = _make_agg(NCLS)


def _mm1_body(x_ref, w_ref, o_ref):
    o_ref[...] = jnp.dot(x_ref[...], w_ref[...],
                         preferred_element_type=jnp.float32)


def _l2_body(a0_ref, a1_ref, s1_ref, b1_ref, w_ref, o_ref):
    h = jnp.maximum(a0_ref[0] + a1_ref[0] + s1_ref[...] + b1_ref[...], 0.0)
    o_ref[...] = jnp.dot(h, w_ref[...], preferred_element_type=jnp.float32)


def _comb_body(a0_ref, a1_ref, s2_ref, b2_ref, o_ref):
    o_ref[...] = a0_ref[0] + a1_ref[0] + s2_ref[...] + b2_ref[...]


def kernel(features, triples, weights1, bias1, weights2, bias2):
    fr = triples[:, 0]
    rel = triples[:, 1]
    to = triples[:, 2]
    didx = jnp.concatenate([fr * R + rel, to * R + rel + 8])
    gidx = jnp.concatenate([to * R + rel, fr * R + rel + 8])
    sidx = jnp.concatenate([fr, to])
    gidx3 = gidx.reshape(NW, NCHUNK, B)
    sidx3 = sidx.reshape(NW, NCHUNK, B)
    didx3 = didx.reshape(NW, NCHUNK, B)
    w1cat = jnp.transpose(weights1, (1, 0, 2)).reshape(F_IN, R * HID)
    w2cat = jnp.transpose(weights2, (1, 0, 2)).reshape(HID, R * NCLS)
    zeros_deg = jnp.zeros((DEG_PAD,), jnp.float32)
    zeros1 = jnp.zeros((N_PAD, HID), jnp.float32)
    zeros2 = jnp.zeros((N_PAD, NCLS), jnp.float32)

    degp = _deg_kernel(didx, zeros_deg).reshape(NC, DEG_PAD)
    deg = degp[0] + degp[1]

    nb = N // _MB
    table1 = pl.pallas_call(
        _mm1_body,
        grid=(nb,),
        in_specs=[pl.BlockSpec((_MB, F_IN), lambda i: (i, 0)),
                  pl.BlockSpec((F_IN, R * HID), lambda i: (0, 0))],
        out_specs=pl.BlockSpec((_MB, R * HID), lambda i: (i, 0)),
        out_shape=jax.ShapeDtypeStruct((N, R * HID), jnp.float32),
    )(features, w1cat)

    acc1p = _agg64(table1.reshape(N * R, HID), gidx3, sidx3, didx3, deg, zeros1)

    self1 = table1[:, 16 * HID:]
    table2 = pl.pallas_call(
        _l2_body,
        grid=(nb,),
        in_specs=[pl.BlockSpec((1, _MB, HID), lambda i: (0, i, 0)),
                  pl.BlockSpec((1, _MB, HID), lambda i: (1, i, 0)),
                  pl.BlockSpec((_MB, HID), lambda i: (i, 0)),
                  pl.BlockSpec((1, HID), lambda i: (0, 0)),
                  pl.BlockSpec((HID, R * NCLS), lambda i: (0, 0))],
        out_specs=pl.BlockSpec((_MB, R * NCLS), lambda i: (i, 0)),
        out_shape=jax.ShapeDtypeStruct((N, R * NCLS), jnp.float32),
    )(acc1p, acc1p, self1, bias1.reshape(1, HID), w2cat)

    acc2p = _agg16(table2.reshape(N * R, NCLS), gidx3, sidx3, didx3, deg, zeros2)

    self2 = table2[:, 16 * NCLS:]
    out = pl.pallas_call(
        _comb_body,
        grid=(nb,),
        in_specs=[pl.BlockSpec((1, _MB, NCLS), lambda i: (0, i, 0)),
                  pl.BlockSpec((1, _MB, NCLS), lambda i: (1, i, 0)),
                  pl.BlockSpec((_MB, NCLS), lambda i: (i, 0)),
                  pl.BlockSpec((1, NCLS), lambda i: (0, 0))],
        out_specs=pl.BlockSpec((_MB, NCLS), lambda i: (i, 0)),
        out_shape=jax.ShapeDtypeStruct((N, NCLS), jnp.float32),
    )(acc2p, acc2p, self2, bias2.reshape(1, NCLS))
    return out
